# baseline (speedup 1.0000x reference)
import functools

import jax
import jax.numpy as jnp
from jax import lax
from jax.experimental import pallas as pl
from jax.experimental.pallas import tpu as pltpu
from jax.experimental.pallas import tpu_sc as plsc

_MU = 3.5
_BATCH = 16384
_D = 32
_LANES = 16
_CHUNK = 128


@functools.lru_cache(maxsize=1)
def _build():
    info = plsc.get_sparse_core_info()
    nc, ns = info.num_cores, info.num_subcores
    nw = nc * ns
    bpw = _BATCH // nw
    nch = bpw // _CHUNK
    ngrp = bpw // _LANES

    mesh = plsc.VectorSubcoreMesh(core_axis_name="c", subcore_axis_name="s")

    @functools.partial(
        pl.kernel,
        out_type=jax.ShapeDtypeStruct((nw, bpw), jnp.float32),
        mesh=mesh,
        compiler_params=pltpu.CompilerParams(
            needs_layout_passes=False, use_tc_tiling_on_sc=False),
        scratch_types=[
            pltpu.VMEM((nch, _CHUNK), jnp.int32),
            pltpu.VMEM((nch, _CHUNK), jnp.int32),
            pltpu.VMEM((bpw, _D), jnp.float32),
            pltpu.VMEM((bpw, _D), jnp.float32),
            pltpu.VMEM((bpw,), jnp.float32),
            pltpu.VMEM((bpw,), jnp.float32),
            pltpu.VMEM((bpw,), jnp.float32),
            pltpu.SemaphoreType.DMA,
        ],
    )
    def mf_kernel(u_hbm, v_hbm, ue_hbm, ie_hbm, ub_hbm, ib_hbm, out_hbm,
                  u_idx, v_idx, ue_v, ie_v, ub_v, ib_v, out_v, sem):
        wid = lax.axis_index("s") * nc + lax.axis_index("c")
        pltpu.sync_copy(u_hbm.at[wid], u_idx)
        pltpu.sync_copy(v_hbm.at[wid], v_idx)

        copies = []
        for j in range(nch):
            sl = pl.ds(j * _CHUNK, _CHUNK)
            copies.append(pltpu.async_copy(ue_hbm.at[u_idx.at[j]], ue_v.at[sl], sem))
            copies.append(pltpu.async_copy(ie_hbm.at[v_idx.at[j]], ie_v.at[sl], sem))
            copies.append(pltpu.async_copy(ub_hbm.at[u_idx.at[j]], ub_v.at[sl], sem))
            copies.append(pltpu.async_copy(ib_hbm.at[v_idx.at[j]], ib_v.at[sl], sem))
        for c in copies:
            c.wait()

        lane = lax.iota(jnp.int32, _LANES)

        def group(g, carry):
            base = g * _LANES
            acc = ub_v[pl.ds(base, _LANES)] + ib_v[pl.ds(base, _LANES)] + _MU
            rows = base + lane
            for d in range(_D):
                cols = jnp.full((_LANES,), d, jnp.int32)
                acc = acc + (plsc.load_gather(ue_v, [rows, cols])
                             * plsc.load_gather(ie_v, [rows, cols]))
            out_v[pl.ds(base, _LANES)] = acc
            return carry

        lax.fori_loop(0, ngrp, group, 0)
        pltpu.sync_copy(out_v, out_hbm.at[wid])

    return mf_kernel, nw, bpw, nch


def kernel(u, v, user_embd, item_embd, user_bias, item_bias):
    mf_kernel, nw, bpw, nch = _build()
    u3 = u.reshape(nw, nch, _CHUNK)
    v3 = v.reshape(nw, nch, _CHUNK)
    ub = user_bias.reshape(-1)
    ib = item_bias.reshape(-1)
    out = mf_kernel(u3, v3, user_embd, item_embd, ub, ib)
    return out.reshape(-1)
